# Initial kernel scaffold; baseline (speedup 1.0000x reference)
import jax
import jax.numpy as jnp
from jax.experimental import pallas as pl


def kernel(inputs, embedding):
    raise NotImplementedError("write your pallas kernel here")


# speedup vs baseline: 1.0137x; 1.0137x over previous
import functools

import jax
import jax.numpy as jnp
from jax import lax
from jax.experimental import pallas as pl
from jax.experimental.pallas import tpu as pltpu
from jax.experimental.pallas import tpu_sc as plsc

N_EMB = 8192
DIM = 256
N_ROWS = 16384
XB = 2048
EB = 2048
NX = N_ROWS // XB
NE = N_EMB // EB
LOSS_SCALE = 1.25 / (N_ROWS * DIM)
WIN_B1 = 2736
WIN_B2 = 5472


def _argmin_body(xb_ref, eb_ref, xsq_ref, esq_ref, idx_ref, loss_ref,
                 minval, minidx):
    j = pl.program_id(1)

    dots = lax.dot_general(
        xb_ref[...], eb_ref[...], (((1,), (1,)), ((), ())),
        preferred_element_type=jnp.float32,
    )
    scores = (xsq_ref[...] + esq_ref[...]) - 2.0 * dots

    col = lax.broadcasted_iota(jnp.int32, scores.shape, 1) + j * EB
    bnd = jnp.where(j <= 1, WIN_B1, WIN_B2)
    inf = jnp.float32(jnp.inf)

    @pl.when(j == 0)
    def _():
        minval[...] = jnp.full_like(minval, jnp.inf)
        minidx[...] = jnp.zeros_like(minidx)

    def merge(s):
        lm = jnp.min(s, axis=1, keepdims=True)
        la = jnp.min(jnp.where(s == lm, col, N_EMB), axis=1, keepdims=True)
        upd = lm < minval[...]
        minidx[...] = jnp.where(upd, la, minidx[...])
        minval[...] = jnp.where(upd, lm, minval[...])

    @pl.when(jnp.logical_or(j == 0, j == NE - 1))
    def _():
        merge(scores)

    @pl.when(jnp.logical_and(j > 0, j < NE - 1))
    def _():
        merge(jnp.where(col < bnd, scores, inf))
        minval[...] = minval[...].astype(jnp.bfloat16).astype(jnp.float32)
        merge(jnp.where(col >= bnd, scores, inf))

    @pl.when(j == NE - 1)
    def _():
        idx_ref[...] = minidx[...]
        i = pl.program_id(0)

        @pl.when(i == 0)
        def _():
            loss_ref[...] = jnp.zeros_like(loss_ref)

        loss_ref[...] += jnp.sum(minval[...]).reshape(1, 1)

        @pl.when(i == NX - 1)
        def _():
            loss_ref[...] = loss_ref[...] * LOSS_SCALE


def _tc_argmin(xb16, eb16, xsq2d, esq2d):
    return pl.pallas_call(
        _argmin_body,
        grid=(NX, NE),
        in_specs=[
            pl.BlockSpec((XB, DIM), lambda i, j: (i, 0)),
            pl.BlockSpec((EB, DIM), lambda i, j: (j, 0)),
            pl.BlockSpec((XB, 1), lambda i, j: (i, 0)),
            pl.BlockSpec((1, EB), lambda i, j: (0, j)),
        ],
        out_specs=[
            pl.BlockSpec((XB, 1), lambda i, j: (i, 0)),
            pl.BlockSpec((1, 1), lambda i, j: (0, 0)),
        ],
        out_shape=[
            jax.ShapeDtypeStruct((N_ROWS, 1), jnp.int32),
            jax.ShapeDtypeStruct((1, 1), jnp.float32),
        ],
        scratch_shapes=[
            pltpu.VMEM((XB, 1), jnp.float32),
            pltpu.VMEM((XB, 1), jnp.int32),
        ],
        compiler_params=pltpu.CompilerParams(
            dimension_semantics=("arbitrary", "arbitrary"),
        ),
    )(xb16, eb16, xsq2d, esq2d)


def _sc_gather(embedding, idx_flat):
    info = plsc.get_sparse_core_info()
    nw = info.num_cores * info.num_subcores
    b_per_w = N_ROWS // nw
    ch = 128
    n_ch = b_per_w // ch
    mesh = plsc.VectorSubcoreMesh(core_axis_name="c", subcore_axis_name="s")

    @functools.partial(
        pl.kernel,
        mesh=mesh,
        out_type=jax.ShapeDtypeStruct((N_ROWS, DIM), jnp.float32),
        scratch_types=[
            pltpu.VMEM((ch,), jnp.int32),
            pltpu.VMEM((ch, DIM), jnp.float32),
            pltpu.SemaphoreType.DMA,
        ],
    )
    def gather_k(table_hbm, idx_hbm, out_hbm, idx_v, rows_v, sem):
        wid = lax.axis_index("s") * info.num_cores + lax.axis_index("c")
        base = wid * b_per_w

        def body(c, carry):
            off = base + c * ch
            pltpu.sync_copy(idx_hbm.at[pl.ds(off, ch)], idx_v)
            pltpu.async_copy(table_hbm.at[idx_v], rows_v, sem).wait()
            pltpu.sync_copy(rows_v, out_hbm.at[pl.ds(off, ch)])
            return carry

        lax.fori_loop(0, n_ch, body, 0)

    return gather_k(embedding, idx_flat)


def kernel(inputs, embedding):
    flat = inputs.reshape(-1, DIM)
    xsq2d = jnp.sum(inputs * inputs, axis=2).reshape(N_ROWS, 1)
    esq2d = jnp.sum(embedding * embedding, axis=1).reshape(1, N_EMB)
    idx2d, loss11 = _tc_argmin(
        flat.astype(jnp.bfloat16), embedding.astype(jnp.bfloat16),
        xsq2d, esq2d)
    quant = _sc_gather(embedding, idx2d.reshape(-1))
    return (quant.reshape(inputs.shape), idx2d, loss11.reshape(()))


# speedup vs baseline: 1.2317x; 1.2152x over previous
import functools

import jax
import jax.numpy as jnp
from jax import lax
from jax.experimental import pallas as pl
from jax.experimental.pallas import tpu as pltpu
from jax.experimental.pallas import tpu_sc as plsc

N_EMB = 8192
DIM = 256
N_ROWS = 16384
XB = 2048
EB = 2048
NX = N_ROWS // XB
NE = N_EMB // EB
LOSS_SCALE = 1.25 / (N_ROWS * DIM)
WIN_B1 = 2736
WIN_B2 = 5472


NG = EB // 128


def _argmin_body(xb_ref, eb_ref, xsq_ref, esq_ref, idx_ref, loss_ref,
                 minval, minidx, vacc, gacc):
    j = pl.program_id(1)

    dots = lax.dot_general(
        xb_ref[...], eb_ref[...], (((1,), (1,)), ((), ())),
        preferred_element_type=jnp.float32,
    )
    scores = (xsq_ref[...] + esq_ref[...]) + dots

    lane = lax.broadcasted_iota(jnp.int32, (XB, 128), 1)
    inf = jnp.float32(jnp.inf)

    @pl.when(j == 0)
    def _():
        minval[...] = jnp.full_like(minval, jnp.inf)
        minidx[...] = jnp.zeros_like(minidx)
        vacc[...] = jnp.full_like(vacc, jnp.inf)
        gacc[...] = jnp.zeros_like(gacc)

    def upd_group(g, mask=None):
        sub = scores[:, g * 128:(g + 1) * 128]
        upd = sub < vacc[...]
        if mask is not None:
            upd = jnp.logical_and(upd, mask)
        gacc[...] = jnp.where(upd, j * NG + g, gacc[...])
        vacc[...] = jnp.where(upd, sub, vacc[...])

    def end_window(do_round):
        wmin = jnp.min(vacc[...], axis=1, keepdims=True)
        colfull = gacc[...] * 128 + lane
        wcol = jnp.min(jnp.where(vacc[...] == wmin, colfull, N_EMB),
                       axis=1, keepdims=True)
        if do_round:
            minval[...] = minval[...].astype(jnp.bfloat16).astype(jnp.float32)
        upd = wmin < minval[...]
        minidx[...] = jnp.where(upd, wcol, minidx[...])
        minval[...] = jnp.where(upd, wmin, minval[...])
        vacc[...] = jnp.full_like(vacc[...], jnp.inf)
        gacc[...] = jnp.zeros_like(gacc[...])

    @pl.when(j == 0)
    def _():
        for g in range(NG):
            upd_group(g)

    @pl.when(j == 1)
    def _():
        for g in range(5):
            upd_group(g)
        upd_group(5, mask=lane < 48)
        end_window(do_round=False)
        upd_group(5, mask=lane >= 48)
        for g in range(6, NG):
            upd_group(g)

    @pl.when(j == 2)
    def _():
        for g in range(10):
            upd_group(g)
        upd_group(10, mask=lane < 96)
        end_window(do_round=True)
        upd_group(10, mask=lane >= 96)
        for g in range(11, NG):
            upd_group(g)

    @pl.when(j == NE - 1)
    def _():
        for g in range(NG):
            upd_group(g)
        end_window(do_round=True)
        idx_ref[...] = minidx[...]
        i = pl.program_id(0)

        @pl.when(i == 0)
        def _():
            loss_ref[...] = jnp.zeros_like(loss_ref)

        loss_ref[...] += jnp.sum(minval[...]).reshape(1, 1)

        @pl.when(i == NX - 1)
        def _():
            loss_ref[...] = loss_ref[...] * LOSS_SCALE


def _tc_argmin(xb16, eb16, xsq2d, esq2d):
    return pl.pallas_call(
        _argmin_body,
        grid=(NX, NE),
        in_specs=[
            pl.BlockSpec((XB, DIM), lambda i, j: (i, 0)),
            pl.BlockSpec((EB, DIM), lambda i, j: (j, 0)),
            pl.BlockSpec((XB, 1), lambda i, j: (i, 0)),
            pl.BlockSpec((1, EB), lambda i, j: (0, j)),
        ],
        out_specs=[
            pl.BlockSpec((XB, 1), lambda i, j: (i, 0)),
            pl.BlockSpec((1, 1), lambda i, j: (0, 0)),
        ],
        out_shape=[
            jax.ShapeDtypeStruct((N_ROWS, 1), jnp.int32),
            jax.ShapeDtypeStruct((1, 1), jnp.float32),
        ],
        scratch_shapes=[
            pltpu.VMEM((XB, 1), jnp.float32),
            pltpu.VMEM((XB, 1), jnp.int32),
            pltpu.VMEM((XB, 128), jnp.float32),
            pltpu.VMEM((XB, 128), jnp.int32),
        ],
        compiler_params=pltpu.CompilerParams(
            dimension_semantics=("arbitrary", "arbitrary"),
        ),
    )(xb16, eb16, xsq2d, esq2d)


def _sc_gather(embedding, idx_flat):
    info = plsc.get_sparse_core_info()
    nw = info.num_cores * info.num_subcores
    b_per_w = N_ROWS // nw
    ch = 128
    n_ch = b_per_w // ch
    mesh = plsc.VectorSubcoreMesh(core_axis_name="c", subcore_axis_name="s")

    @functools.partial(
        pl.kernel,
        mesh=mesh,
        out_type=jax.ShapeDtypeStruct((N_ROWS, DIM), jnp.float32),
        scratch_types=[
            pltpu.VMEM((ch,), jnp.int32),
            pltpu.VMEM((ch, DIM), jnp.float32),
            pltpu.SemaphoreType.DMA,
        ],
    )
    def gather_k(table_hbm, idx_hbm, out_hbm, idx_v, rows_v, sem):
        wid = lax.axis_index("s") * info.num_cores + lax.axis_index("c")
        base = wid * b_per_w

        def body(c, carry):
            off = base + c * ch
            pltpu.sync_copy(idx_hbm.at[pl.ds(off, ch)], idx_v)
            pltpu.async_copy(table_hbm.at[idx_v], rows_v, sem).wait()
            pltpu.sync_copy(rows_v, out_hbm.at[pl.ds(off, ch)])
            return carry

        lax.fori_loop(0, n_ch, body, 0)

    return gather_k(embedding, idx_flat)


def kernel(inputs, embedding):
    flat = inputs.reshape(-1, DIM)
    xsq2d = jnp.sum(inputs * inputs, axis=2).reshape(N_ROWS, 1)
    esq2d = jnp.sum(embedding * embedding, axis=1).reshape(1, N_EMB)
    idx2d, loss11 = _tc_argmin(
        (flat * -2.0).astype(jnp.bfloat16), embedding.astype(jnp.bfloat16),
        xsq2d, esq2d)
    quant = _sc_gather(embedding, idx2d.reshape(-1))
    return (quant.reshape(inputs.shape), idx2d, loss11.reshape(()))
